# baseline (speedup 1.0000x reference)
import functools
import math

import jax
import jax.numpy as jnp
from jax import lax
from jax.experimental import pallas as pl
from jax.experimental.pallas import tpu as pltpu
from jax.experimental.pallas import tpu_sc as plsc

_HD = 64
_K = 8
_EPS = 1e-8
_IMIN = -2**31
_IDXMASK = 0x3FFF
_KEYMASK = ~0x3FFF


def _norm_body(km_ref, kn_ref):
    x = km_ref[0]
    n = jnp.sqrt(jnp.sum(x * x, axis=-1, keepdims=True))
    kn_ref[0] = (x / jnp.maximum(n, _EPS)).astype(jnp.bfloat16)


def _normalize_keys(key_memories):
    H, M, D = key_memories.shape
    return pl.pallas_call(
        _norm_body,
        grid=(H,),
        in_specs=[pl.BlockSpec((1, M, D), lambda h: (h, 0, 0))],
        out_specs=pl.BlockSpec((1, M, D), lambda h: (h, 0, 0)),
        out_shape=jax.ShapeDtypeStruct((H, M, D), jnp.bfloat16),
    )(key_memories)


_SORT8 = [(0, 1), (2, 3), (4, 5), (6, 7),
          (0, 2), (1, 3), (4, 6), (5, 7),
          (1, 2), (5, 6),
          (0, 4), (1, 5), (2, 6), (3, 7),
          (2, 4), (3, 5),
          (1, 2), (3, 4), (5, 6)]
_BITONIC8 = [(0, 4), (1, 5), (2, 6), (3, 7),
             (0, 2), (1, 3), (4, 6), (5, 7),
             (0, 1), (2, 3), (4, 5), (6, 7)]
_BIAS = 0.5


def _apply_network(s, net):
    for i, j in net:
        a, b = s[i], s[j]
        s[i] = jnp.maximum(a, b)
        s[j] = jnp.minimum(a, b)
    return s


def _merge_sorted8(a, b):
    c = [jnp.maximum(a[i], b[7 - i]) for i in range(_K)]
    return _apply_network(c, _BITONIC8)


def _topk_body(q_ref, kn_ref, scores_ref, idx_ref, run_ref, *, mb, mtot, h0):
    h = pl.program_id(0)
    mt = pl.program_id(2)
    nmt = pl.num_programs(2)

    q = q_ref[0]
    scale = 1.0 / math.sqrt(_HD)
    qn = (q * scale) / jnp.maximum(
        jnp.sqrt(jnp.sum(q * q, axis=-1, keepdims=True)), _EPS)
    st = lax.dot_general(
        kn_ref[0], qn.astype(jnp.bfloat16),
        (((1,), (1,)), ((), ())),
        preferred_element_type=jnp.float32,
    )

    b = lax.bitcast_convert_type(st + _BIAS, jnp.int32)
    gidx = mt * mb + lax.broadcasted_iota(jnp.int32, st.shape, 0)
    keys = lax.bitcast_convert_type((b & _KEYMASK) | gidx, jnp.float32)

    n = mb // _K
    s = [keys[i * n:(i + 1) * n, :] for i in range(_K)]
    s = _apply_network(s, _SORT8)
    while n > 1:
        half = n // 2
        a = [x[:half, :] for x in s]
        bb = [x[half:, :] for x in s]
        s = _merge_sorted8(a, bb)
        n = half

    def emit(newrun_arr):
        ki = lax.bitcast_convert_type(newrun_arr, jnp.int32)
        val = lax.bitcast_convert_type(ki & _KEYMASK, jnp.float32)
        scores_ref[0] = val - _BIAS
        idx_ref[0] = (ki & _IDXMASK) + (h + h0) * mtot

    if mtot == mb:
        emit(jnp.concatenate(s, axis=0))
    else:
        run_old = run_ref[...]
        run = [jnp.where(mt == 0, -1.0, run_old[i:i + 1, :]) for i in range(_K)]
        newrun = _merge_sorted8(run, s)
        newrun_arr = jnp.concatenate(newrun, axis=0)
        run_ref[...] = newrun_arr

        @pl.when(mt == nmt - 1)
        def _():
            emit(newrun_arr)


def _topk(q_r, kn, h0=0, qt=256, mb=16384):
    H, Q, D = q_r.shape
    M = kn.shape[1]
    body = functools.partial(_topk_body, mb=mb, mtot=M, h0=h0)
    return pl.pallas_call(
        body,
        grid=(H, Q // qt, M // mb),
        in_specs=[
            pl.BlockSpec((1, qt, D), lambda h, q, m: (h, q, 0)),
            pl.BlockSpec((1, mb, D), lambda h, q, m: (h, m, 0)),
        ],
        out_specs=[
            pl.BlockSpec((1, _K, qt), lambda h, q, m: (h, 0, q)),
            pl.BlockSpec((1, _K, qt), lambda h, q, m: (h, 0, q)),
        ],
        out_shape=[
            jax.ShapeDtypeStruct((H, _K, Q), jnp.float32),
            jax.ShapeDtypeStruct((H, _K, Q), jnp.int32),
        ],
        scratch_shapes=[pltpu.VMEM((_K, qt), jnp.float32)],
        compiler_params=pltpu.CompilerParams(
            dimension_semantics=("arbitrary", "arbitrary", "arbitrary"),
        ),
    )(q_r, kn)


def _sc_gather(table, idx3):
    R, D = table.shape
    NW, n_chunks, CH = idx3.shape
    mesh = plsc.VectorSubcoreMesh(core_axis_name="c", subcore_axis_name="s")

    @functools.partial(
        pl.kernel,
        mesh=mesh,
        out_type=jax.ShapeDtypeStruct((NW * n_chunks * CH, D), jnp.float32),
        scratch_types=[
            pltpu.VMEM((n_chunks, CH), jnp.int32),
            pltpu.VMEM((CH, D), jnp.float32),
            pltpu.VMEM((CH, D), jnp.float32),
            pltpu.SemaphoreType.DMA,
            pltpu.SemaphoreType.DMA,
        ],
        compiler_params=pltpu.CompilerParams(use_tc_tiling_on_sc=False),
    )
    def k(table_hbm, idx_hbm, out_hbm, idx_v, rows0, rows1, sem0, sem1):
        wid = lax.axis_index("s") * 2 + lax.axis_index("c")
        base = wid * n_chunks
        pltpu.sync_copy(idx_hbm.at[wid], idx_v)
        pltpu.async_copy(table_hbm.at[idx_v.at[0]], rows0, sem0)

        def pair(i, _):
            cc0 = 2 * i
            pltpu.async_copy(table_hbm.at[idx_v.at[cc0 + 1]], rows1, sem1)
            pltpu.make_async_copy(table_hbm.at[idx_v.at[cc0]], rows0, sem0).wait()
            pltpu.sync_copy(rows0, out_hbm.at[pl.ds((base + cc0) * CH, CH)])

            @pl.when(i < n_chunks // 2 - 1)
            def _():
                pltpu.async_copy(table_hbm.at[idx_v.at[cc0 + 2]], rows0, sem0)
            pltpu.make_async_copy(table_hbm.at[idx_v.at[cc0 + 1]], rows1, sem1).wait()
            pltpu.sync_copy(rows1, out_hbm.at[pl.ds((base + cc0 + 1) * CH, CH)])
            return 0

        lax.fori_loop(0, n_chunks // 2, pair, 0)

    return k(table, idx3)


def _combine_body(gate_ref, s_ref, rows_ref, o_ref, out_ref):
    g = 1.0 / (1.0 + jnp.exp(-gate_ref[pl.program_id(0), 0]))
    wm = jnp.zeros(out_ref.shape[1:], jnp.float32)
    for j in range(_K):
        wm = wm + rows_ref[0, j] * s_ref[0, :, j:j + 1]
    out_ref[0] = g * wm + (1.0 - g) * o_ref[0]


def _combine(gate, scores, rows4, o_r, qt=512):
    H, _, Q, D = rows4.shape
    return pl.pallas_call(
        _combine_body,
        grid=(H, Q // qt),
        in_specs=[
            pl.BlockSpec((H, 1), lambda h, q: (0, 0), memory_space=pltpu.SMEM),
            pl.BlockSpec((1, qt, _K), lambda h, q: (h, q, 0)),
            pl.BlockSpec((1, _K, qt, D), lambda h, q: (h, 0, q, 0)),
            pl.BlockSpec((1, qt, D), lambda h, q: (h, q, 0)),
        ],
        out_specs=pl.BlockSpec((1, qt, D), lambda h, q: (h, q, 0)),
        out_shape=jax.ShapeDtypeStruct((H, Q, D), jnp.float32),
    )(gate.reshape(H, 1), scores, rows4, o_r)


def kernel(inputs, query, key, value, outputs, gate, key_memories, value_memories):
    B, H, S, hd = query.shape
    Q = B * S
    M = key_memories.shape[1]

    q_r = jnp.transpose(query, (1, 0, 2, 3)).reshape(H, Q, hd)
    o_r = jnp.transpose(outputs, (1, 0, 2, 3)).reshape(H, Q, hd)

    kn = _normalize_keys(key_memories)
    vm_flat = value_memories.reshape(H * M, hd)
    NW, CH = 32, 128

    Hh = H // 2
    outs = []
    parts = []
    for g in range(2):
        lo = g * Hh
        scores_t, idx_t = _topk(q_r[lo:lo + Hh], kn[lo:lo + Hh], h0=lo)
        parts.append((lo, jnp.transpose(scores_t, (0, 2, 1)), idx_t))
    for lo, scores, idx_t in parts:
        n_chunks = (Hh * Q * _K) // (NW * CH)
        rows = _sc_gather(vm_flat, idx_t.reshape(NW, n_chunks, CH))
        outs.append(_combine(gate[lo:lo + Hh], scores,
                             rows.reshape(Hh, _K, Q, hd), o_r[lo:lo + Hh]))
    out = jnp.concatenate(outs, axis=0)
    return jnp.transpose(out.reshape(H, B, S, hd), (1, 0, 2, 3))


# speedup vs baseline: 1.0276x; 1.0276x over previous
import functools
import math

import jax
import jax.numpy as jnp
from jax import lax
from jax.experimental import pallas as pl
from jax.experimental.pallas import tpu as pltpu
from jax.experimental.pallas import tpu_sc as plsc

_HD = 64
_K = 8
_EPS = 1e-8
_IMIN = -2**31
_IDXMASK = 0x3FFF
_KEYMASK = ~0x3FFF


def _norm_body(km_ref, kn_ref):
    x = km_ref[0]
    n = jnp.sqrt(jnp.sum(x * x, axis=-1, keepdims=True))
    kn_ref[0] = (x / jnp.maximum(n, _EPS)).astype(jnp.bfloat16)


def _normalize_keys(key_memories):
    H, M, D = key_memories.shape
    return pl.pallas_call(
        _norm_body,
        grid=(H,),
        in_specs=[pl.BlockSpec((1, M, D), lambda h: (h, 0, 0))],
        out_specs=pl.BlockSpec((1, M, D), lambda h: (h, 0, 0)),
        out_shape=jax.ShapeDtypeStruct((H, M, D), jnp.bfloat16),
    )(key_memories)


_SORT8 = [(0, 1), (2, 3), (4, 5), (6, 7),
          (0, 2), (1, 3), (4, 6), (5, 7),
          (1, 2), (5, 6),
          (0, 4), (1, 5), (2, 6), (3, 7),
          (2, 4), (3, 5),
          (1, 2), (3, 4), (5, 6)]
_BITONIC8 = [(0, 4), (1, 5), (2, 6), (3, 7),
             (0, 2), (1, 3), (4, 6), (5, 7),
             (0, 1), (2, 3), (4, 5), (6, 7)]
_BIAS = 0.5


def _apply_network(s, net):
    for i, j in net:
        a, b = s[i], s[j]
        s[i] = jnp.maximum(a, b)
        s[j] = jnp.minimum(a, b)
    return s


def _merge_sorted8(a, b):
    c = [jnp.maximum(a[i], b[7 - i]) for i in range(_K)]
    return _apply_network(c, _BITONIC8)


def _topk_body(q_ref, kn_ref, scores_ref, idx_ref, run_ref, *, mb, mtot, h0):
    h = pl.program_id(0)
    mt = pl.program_id(2)
    nmt = pl.num_programs(2)

    q = q_ref[0]
    scale = 1.0 / math.sqrt(_HD)
    qn = (q * scale) / jnp.maximum(
        jnp.sqrt(jnp.sum(q * q, axis=-1, keepdims=True)), _EPS)
    st = lax.dot_general(
        kn_ref[0], qn.astype(jnp.bfloat16),
        (((1,), (1,)), ((), ())),
        preferred_element_type=jnp.float32,
    )

    b = lax.bitcast_convert_type(st + _BIAS, jnp.int32)
    gidx = mt * mb + lax.broadcasted_iota(jnp.int32, st.shape, 0)
    keys = lax.bitcast_convert_type((b & _KEYMASK) | gidx, jnp.float32)

    n = mb // _K
    s = [keys[i * n:(i + 1) * n, :] for i in range(_K)]
    s = _apply_network(s, _SORT8)
    while n > 1:
        half = n // 2
        a = [x[:half, :] for x in s]
        bb = [x[half:, :] for x in s]
        s = _merge_sorted8(a, bb)
        n = half

    def emit(newrun_arr):
        ki = lax.bitcast_convert_type(newrun_arr, jnp.int32)
        val = lax.bitcast_convert_type(ki & _KEYMASK, jnp.float32)
        scores_ref[0] = val - _BIAS
        idx_ref[0] = (ki & _IDXMASK) + (h + h0) * mtot

    if mtot == mb:
        emit(jnp.concatenate(s, axis=0))
    else:
        run_old = run_ref[...]
        run = [jnp.where(mt == 0, -1.0, run_old[i:i + 1, :]) for i in range(_K)]
        newrun = _merge_sorted8(run, s)
        newrun_arr = jnp.concatenate(newrun, axis=0)
        run_ref[...] = newrun_arr

        @pl.when(mt == nmt - 1)
        def _():
            emit(newrun_arr)


def _topk(q_r, kn, h0=0, qt=256, mb=16384):
    H, Q, D = q_r.shape
    M = kn.shape[1]
    body = functools.partial(_topk_body, mb=mb, mtot=M, h0=h0)
    return pl.pallas_call(
        body,
        grid=(H, Q // qt, M // mb),
        in_specs=[
            pl.BlockSpec((1, qt, D), lambda h, q, m: (h, q, 0)),
            pl.BlockSpec((1, mb, D), lambda h, q, m: (h, m, 0)),
        ],
        out_specs=[
            pl.BlockSpec((1, _K, qt), lambda h, q, m: (h, 0, q)),
            pl.BlockSpec((1, _K, qt), lambda h, q, m: (h, 0, q)),
        ],
        out_shape=[
            jax.ShapeDtypeStruct((H, _K, Q), jnp.float32),
            jax.ShapeDtypeStruct((H, _K, Q), jnp.int32),
        ],
        scratch_shapes=[pltpu.VMEM((_K, qt), jnp.float32)],
        compiler_params=pltpu.CompilerParams(
            dimension_semantics=("arbitrary", "arbitrary", "arbitrary"),
        ),
    )(q_r, kn)


def _sc_gather(table, idx3):
    R, D = table.shape
    NW, n_chunks, CH = idx3.shape
    mesh = plsc.VectorSubcoreMesh(core_axis_name="c", subcore_axis_name="s")

    @functools.partial(
        pl.kernel,
        mesh=mesh,
        out_type=jax.ShapeDtypeStruct((NW * n_chunks * CH, D), jnp.float32),
        scratch_types=[
            pltpu.VMEM((n_chunks, CH), jnp.int32),
            pltpu.VMEM((CH, D), jnp.float32),
            pltpu.VMEM((CH, D), jnp.float32),
            pltpu.SemaphoreType.DMA,
            pltpu.SemaphoreType.DMA,
        ],
        compiler_params=pltpu.CompilerParams(use_tc_tiling_on_sc=False),
    )
    def k(table_hbm, idx_hbm, out_hbm, idx_v, rows0, rows1, sem0, sem1):
        wid = lax.axis_index("s") * 2 + lax.axis_index("c")
        base = wid * n_chunks
        pltpu.sync_copy(idx_hbm.at[wid], idx_v)
        pltpu.async_copy(table_hbm.at[idx_v.at[0]], rows0, sem0)

        def pair(i, _):
            cc0 = 2 * i
            pltpu.async_copy(table_hbm.at[idx_v.at[cc0 + 1]], rows1, sem1)
            pltpu.make_async_copy(table_hbm.at[idx_v.at[cc0]], rows0, sem0).wait()
            pltpu.sync_copy(rows0, out_hbm.at[pl.ds((base + cc0) * CH, CH)])

            @pl.when(i < n_chunks // 2 - 1)
            def _():
                pltpu.async_copy(table_hbm.at[idx_v.at[cc0 + 2]], rows0, sem0)
            pltpu.make_async_copy(table_hbm.at[idx_v.at[cc0 + 1]], rows1, sem1).wait()
            pltpu.sync_copy(rows1, out_hbm.at[pl.ds((base + cc0 + 1) * CH, CH)])
            return 0

        lax.fori_loop(0, n_chunks // 2, pair, 0)

    return k(table, idx3)


def _combine_body(gate_ref, s_ref, rows_ref, o_ref, out_ref):
    g = 1.0 / (1.0 + jnp.exp(-gate_ref[pl.program_id(0), 0]))
    wm = jnp.zeros(out_ref.shape[1:], jnp.float32)
    for j in range(_K):
        wm = wm + rows_ref[0, j] * s_ref[0, :, j:j + 1]
    out_ref[0] = g * wm + (1.0 - g) * o_ref[0]


def _combine(gate, scores, rows4, o_r, qt=512):
    H, _, Q, D = rows4.shape
    return pl.pallas_call(
        _combine_body,
        grid=(H, Q // qt),
        in_specs=[
            pl.BlockSpec((H, 1), lambda h, q: (0, 0), memory_space=pltpu.SMEM),
            pl.BlockSpec((1, qt, _K), lambda h, q: (h, q, 0)),
            pl.BlockSpec((1, _K, qt, D), lambda h, q: (h, 0, q, 0)),
            pl.BlockSpec((1, qt, D), lambda h, q: (h, q, 0)),
        ],
        out_specs=pl.BlockSpec((1, qt, D), lambda h, q: (h, q, 0)),
        out_shape=jax.ShapeDtypeStruct((H, Q, D), jnp.float32),
    )(gate.reshape(H, 1), scores, rows4, o_r)


def kernel(inputs, query, key, value, outputs, gate, key_memories, value_memories):
    B, H, S, hd = query.shape
    Q = B * S
    M = key_memories.shape[1]

    q_r = jnp.transpose(query, (1, 0, 2, 3)).reshape(H, Q, hd)
    o_r = jnp.transpose(outputs, (1, 0, 2, 3)).reshape(H, Q, hd)

    kn = _normalize_keys(key_memories)
    scores_t, idx_t = _topk(q_r, kn)
    scores = jnp.transpose(scores_t, (0, 2, 1))

    NW, CH = 32, 128
    n_chunks = (H * Q * _K) // (NW * CH)
    idx3 = idx_t.reshape(NW, n_chunks, CH)
    rows = _sc_gather(value_memories.reshape(H * M, hd), idx3)

    out = _combine(gate, scores, rows.reshape(H, _K, Q, hd), o_r)
    return jnp.transpose(out.reshape(H, B, S, hd), (1, 0, 2, 3))
